# baseline (speedup 1.0000x reference)
import functools

import jax
import jax.numpy as jnp
import numpy as np
from jax import lax
from jax.experimental import pallas as pl
from jax.experimental.pallas import tpu as pltpu
from jax.experimental.pallas import tpu_sc as plsc

N_USERS = 10000
NP = 10240
EMB = 256
HALF = 128
E_INC = 160000
CH = 128
EP = 163840
EPC = EP // CH
PAD_ROW = 10200
NTILE = 16
ZROWS = NP // NTILE
KTILE = EPC // NTILE
BATCH = 1024
SEQ = 50
NTOK = BATCH * SEQ
GCHUNKS = NTOK // CH


def _make_pe(L, d):
    position = np.arange(L)[:, None].astype(np.float32)
    div_term = np.exp(np.arange(0, d, 2).astype(np.float32) * -(np.log(10000.0) / d))
    pe = np.zeros((L, d), dtype=np.float32)
    pe[:, 0::2] = np.sin(position * div_term)
    pe[:, 1::2] = np.cos(position * div_term)
    return pe


def _seg_body(tab, gidx, sidx, zrows, out,
              gidx_v, sidx_v, gidx2_v, sidx2_v, rows_v, rows2_v,
              accum_sh, sem, sem2):
    cid = lax.axis_index("c")
    sid = lax.axis_index("s")

    pltpu.sync_copy(zrows, rows_v)
    for z in range(ZROWS // CH):
        start = sid * ZROWS + z * CH
        pltpu.sync_copy(rows_v, accum_sh.at[pl.ds(start, CH)])
    plsc.subcore_barrier()

    base_off = cid * NP

    def chunk(k, _):
        off_a = (sid * KTILE + 2 * k) * CH
        off_b = off_a + CH
        pltpu.sync_copy(gidx.at[pl.ds(off_a, CH)], gidx_v)
        pltpu.sync_copy(sidx.at[pl.ds(off_a, CH)], sidx_v)
        pltpu.sync_copy(gidx.at[pl.ds(off_b, CH)], gidx2_v)
        pltpu.sync_copy(sidx.at[pl.ds(off_b, CH)], sidx2_v)

        def prep(t, _):
            gidx_v[pl.ds(t * 16, 16)] = gidx_v[pl.ds(t * 16, 16)] + base_off
            gidx2_v[pl.ds(t * 16, 16)] = gidx2_v[pl.ds(t * 16, 16)] + base_off
            return 0
        lax.fori_loop(0, CH // 16, prep, 0)

        ha = pltpu.async_copy(tab.at[gidx_v], rows_v, sem)
        hb = pltpu.async_copy(tab.at[gidx2_v], rows2_v, sem2)
        ha.wait()
        pltpu.sync_copy(rows_v, accum_sh.at[sidx_v], add=True)
        hb.wait()
        pltpu.sync_copy(rows2_v, accum_sh.at[sidx2_v], add=True)
        return 0
    lax.fori_loop(0, KTILE // 2, chunk, 0)

    plsc.subcore_barrier()
    for z in range(ZROWS // CH):
        start = sid * ZROWS + z * CH
        pltpu.sync_copy(accum_sh.at[pl.ds(start, CH)], rows_v)
        pltpu.sync_copy(rows_v, out.at[pl.ds(base_off + start, CH)])


def _segment_pass(tab2, gidx, sidx, zrows):
    mesh = plsc.VectorSubcoreMesh(core_axis_name="c", subcore_axis_name="s")
    f = pl.kernel(
        _seg_body,
        mesh=mesh,
        out_type=jax.ShapeDtypeStruct((2 * NP, HALF), jnp.float32),
        scratch_types=[
            pltpu.VMEM((CH,), jnp.int32),
            pltpu.VMEM((CH,), jnp.int32),
            pltpu.VMEM((CH,), jnp.int32),
            pltpu.VMEM((CH,), jnp.int32),
            pltpu.VMEM((CH, HALF), jnp.float32),
            pltpu.VMEM((CH, HALF), jnp.float32),
            pltpu.VMEM_SHARED((NP, HALF), jnp.float32),
            pltpu.SemaphoreType.DMA,
            pltpu.SemaphoreType.DMA,
        ],
    )
    return f(tab2, gidx, sidx, zrows)


def _deg_body(gidx, sidx, onesr, zrows, out,
              idx_v, ones_v, rows_v, accum_sh, sem):
    cid = lax.axis_index("c")
    sid = lax.axis_index("s")

    pltpu.sync_copy(onesr, ones_v)
    pltpu.sync_copy(zrows, rows_v)
    for z in range(ZROWS // CH):
        start = sid * ZROWS + z * CH
        pltpu.sync_copy(rows_v, accum_sh.at[pl.ds(start, CH)])
    plsc.subcore_barrier()

    base_off = cid * NP

    def chunk(k, _):
        off = (sid * KTILE + k) * CH

        @pl.when(cid == 0)
        def _():
            pltpu.sync_copy(gidx.at[pl.ds(off, CH)], idx_v)

        @pl.when(cid == 1)
        def _():
            pltpu.sync_copy(sidx.at[pl.ds(off, CH)], idx_v)

        pltpu.sync_copy(ones_v, accum_sh.at[idx_v], add=True)
        return 0
    lax.fori_loop(0, KTILE, chunk, 0)

    plsc.subcore_barrier()
    for z in range(ZROWS // CH):
        start = sid * ZROWS + z * CH
        pltpu.sync_copy(accum_sh.at[pl.ds(start, CH)], rows_v)
        pltpu.sync_copy(rows_v, out.at[pl.ds(base_off + start, CH)])


def _deg_pass(gidx, sidx, onesr, zrows):
    mesh = plsc.VectorSubcoreMesh(core_axis_name="c", subcore_axis_name="s")
    f = pl.kernel(
        _deg_body,
        mesh=mesh,
        out_type=jax.ShapeDtypeStruct((2 * NP, HALF), jnp.float32),
        scratch_types=[
            pltpu.VMEM((CH,), jnp.int32),
            pltpu.VMEM((CH, HALF), jnp.float32),
            pltpu.VMEM((CH, HALF), jnp.float32),
            pltpu.VMEM_SHARED((NP, HALF), jnp.float32),
            pltpu.SemaphoreType.DMA,
        ],
    )
    return f(gidx, sidx, onesr, zrows)


def _gath_body(gtab, titab, idx, grows, tirows, idx_v, rows_v, rows2_v, sem):
    cid = lax.axis_index("c")
    sid = lax.axis_index("s")
    base_off = cid * NP
    out_base = cid * NTOK

    def chunk(k, _):
        off = (sid + k * NTILE) * CH
        pltpu.sync_copy(idx.at[pl.ds(off, CH)], idx_v)

        def prep(t, _):
            idx_v[pl.ds(t * 16, 16)] = idx_v[pl.ds(t * 16, 16)] + base_off
            return 0
        lax.fori_loop(0, CH // 16, prep, 0)

        pltpu.async_copy(gtab.at[idx_v], rows_v, sem).wait()
        pltpu.sync_copy(rows_v, grows.at[pl.ds(out_base + off, CH)])
        pltpu.async_copy(titab.at[idx_v], rows2_v, sem).wait()
        pltpu.sync_copy(rows2_v, tirows.at[pl.ds(out_base + off, CH)])
        return 0
    lax.fori_loop(0, GCHUNKS // NTILE, chunk, 0)


def _example_gather(gtab2, titab2, idx):
    mesh = plsc.VectorSubcoreMesh(core_axis_name="c", subcore_axis_name="s")
    f = pl.kernel(
        _gath_body,
        mesh=mesh,
        out_type=[
            jax.ShapeDtypeStruct((2 * NTOK, HALF), jnp.float32),
            jax.ShapeDtypeStruct((2 * NTOK, HALF), jnp.float32),
        ],
        scratch_types=[
            pltpu.VMEM((CH,), jnp.int32),
            pltpu.VMEM((CH, HALF), jnp.float32),
            pltpu.VMEM((CH, HALF), jnp.float32),
            pltpu.SemaphoreType.DMA,
        ],
    )
    return f(gtab2, titab2, idx)


TR = 1024


def _prep_body(rel, rth, cas, cth, soc, caw, tmp, infl, soch, tmph,
               oxr, oxc, og, oti, osoc, otmp):
    oxr[0] = lax.dot_general(rel[...], rth[...], (((1,), (0,)), ((), ())),
                             preferred_element_type=jnp.float32)
    oxc[0] = lax.dot_general(cas[...], cth[...], (((1,), (0,)), ((), ())),
                             preferred_element_type=jnp.float32)
    og[0] = lax.dot_general(soc[...], caw[...], (((1,), (0,)), ((), ())),
                            preferred_element_type=jnp.float32)
    oti[0] = tmp[...] + infl[...]
    osoc[0] = soch[...]
    otmp[0] = tmph[...]


def _prep(rel_p, rel_theta, cas_p, cas_theta, soc_p, co_attn_wts, tmp_p, infl_p):
    n = NP // TR
    outs = pl.pallas_call(
        _prep_body,
        grid=(2, n),
        in_specs=[
            pl.BlockSpec((TR, EMB), lambda c, i: (i, 0)),
            pl.BlockSpec((EMB, HALF), lambda c, i: (0, c)),
            pl.BlockSpec((TR, EMB), lambda c, i: (i, 0)),
            pl.BlockSpec((EMB, HALF), lambda c, i: (0, c)),
            pl.BlockSpec((TR, EMB), lambda c, i: (i, 0)),
            pl.BlockSpec((EMB, HALF), lambda c, i: (0, c)),
            pl.BlockSpec((TR, HALF), lambda c, i: (i, c)),
            pl.BlockSpec((TR, HALF), lambda c, i: (i, c)),
            pl.BlockSpec((TR, HALF), lambda c, i: (i, c)),
            pl.BlockSpec((TR, HALF), lambda c, i: (i, c)),
        ],
        out_specs=[pl.BlockSpec((1, TR, HALF), lambda c, i: (c, i, 0))] * 6,
        out_shape=[jax.ShapeDtypeStruct((2, NP, HALF), jnp.float32)] * 6,
    )(rel_p, rel_theta, cas_p, cas_theta, soc_p, co_attn_wts, tmp_p, infl_p,
      soc_p, tmp_p)
    return outs


def _scale_body(acc, deg, out):
    d = deg[...]
    inv = jnp.where(d > 0, 1.0 / d, 0.0)
    out[0] = acc[0] * inv


def _scale(acc_st, deg128):
    n = NP // TR
    return pl.pallas_call(
        _scale_body,
        grid=(2, n),
        in_specs=[
            pl.BlockSpec((1, TR, HALF), lambda c, i: (c, i, 0)),
            pl.BlockSpec((TR, HALF), lambda c, i: (i, 0)),
        ],
        out_specs=pl.BlockSpec((1, TR, HALF), lambda c, i: (c, i, 0)),
        out_shape=jax.ShapeDtypeStruct((2, NP, HALF), jnp.float32),
    )(acc_st, deg128)


def _scale_bias_body(acc, deg, bias, out):
    d = deg[...]
    inv = jnp.where(d > 0, 1.0 / d, 0.0)
    out[0] = acc[0] * inv + bias[0]


def _scale_bias(acc_st, deg128, bias_st):
    n = NP // TR
    return pl.pallas_call(
        _scale_bias_body,
        grid=(2, n),
        in_specs=[
            pl.BlockSpec((1, TR, HALF), lambda c, i: (c, i, 0)),
            pl.BlockSpec((TR, HALF), lambda c, i: (i, 0)),
            pl.BlockSpec((1, 1, HALF), lambda c, i: (c, 0, 0)),
        ],
        out_specs=pl.BlockSpec((1, TR, HALF), lambda c, i: (c, i, 0)),
        out_shape=jax.ShapeDtypeStruct((2, NP, HALF), jnp.float32),
    )(acc_st, deg128, bias_st)


TB = 64


def _attn_body(g, ti, mf, pe, hout):
    gA, gB = g[0], g[1]
    tiA, tiB = ti[0], ti[1]
    mf3 = mf[...]
    mf2 = mf[:, :, 0]
    peA = jnp.broadcast_to(pe[0][None], (TB, SEQ, HALF))
    peB = jnp.broadcast_to(pe[1][None], (TB, SEQ, HALF))
    stA = tiA * mf3 + peA
    stB = tiB * mf3 + peB
    score = jnp.sum(gA * stA + gB * stB, axis=2) * mf2
    attn = jnp.tanh(score)
    m = jnp.max(attn, axis=1, keepdims=True)
    e = jnp.exp(attn - m)
    alpha = e / jnp.sum(e, axis=1, keepdims=True)
    a3 = alpha[:, :, None]
    hA = jnp.sum(stA * a3, axis=1)
    hB = jnp.sum(stB * a3, axis=1)
    hout[...] = jnp.concatenate([hA, hB], axis=1)


def _coattn(grows4, tirows4, mf3, pe_st):
    n = BATCH // TB
    return pl.pallas_call(
        _attn_body,
        grid=(n,),
        in_specs=[
            pl.BlockSpec((2, TB, SEQ, HALF), lambda i: (0, i, 0, 0)),
            pl.BlockSpec((2, TB, SEQ, HALF), lambda i: (0, i, 0, 0)),
            pl.BlockSpec((TB, SEQ, 1), lambda i: (i, 0, 0)),
            pl.BlockSpec((2, SEQ, HALF), lambda i: (0, 0, 0)),
        ],
        out_specs=pl.BlockSpec((TB, EMB), lambda i: (i, 0)),
        out_shape=jax.ShapeDtypeStruct((BATCH, EMB), jnp.float32),
    )(grows4, tirows4, mf3, pe_st)


def _pred_body(h_ref, soc_ref, out_ref):
    out_ref[...] = lax.dot_general(
        h_ref[...], soc_ref[...], (((1,), (1,)), ((), ())),
        preferred_element_type=jnp.float32)


def _pred_matmul(h, soc_p):
    TN = 1024
    out = pl.pallas_call(
        _pred_body,
        grid=(NP // TN,),
        in_specs=[
            pl.BlockSpec((BATCH, EMB), lambda i: (0, 0)),
            pl.BlockSpec((TN, EMB), lambda i: (i, 0)),
        ],
        out_specs=pl.BlockSpec((BATCH, TN), lambda i: (0, i)),
        out_shape=jax.ShapeDtypeStruct((BATCH, NP), jnp.float32),
    )(h, soc_p)
    return out[:, :N_USERS]


def _fusion_body(e0, e1, ut, tmp, soc, w1, b1, w2, b2, out):
    i = pl.program_id(0)
    e0A, e0B = e0[0], e0[1]
    e1A, e1B = e1[0], e1[1]
    w1A = w1[:, 0:HALF]
    w1B = w1[:, HALF:EMB]

    def head(eA, eB):
        z = (lax.dot_general(eA, w1A, (((1,), (1,)), ((), ())),
                             preferred_element_type=jnp.float32)
             + lax.dot_general(eB, w1B, (((1,), (1,)), ((), ())),
                               preferred_element_type=jnp.float32))
        t = jnp.tanh(z + b1[...])
        return jnp.sum(t * w2[...], axis=1, keepdims=True) + b2[...]

    w0 = head(e0A, e0B)
    w1s = head(e1A, e1B)
    gate = 2.0 * jax.nn.sigmoid(w0 - w1s)
    dA = ut[0] - tmp[0]
    dB = ut[1] - tmp[1]
    uA = gate * e0A - soc[0]
    uB = gate * e0B - soc[1]
    rows = jnp.sum(dA * dA + dB * dB + uA * uA + uB * uB, axis=1, keepdims=True)
    rowid = i * TR + lax.broadcasted_iota(jnp.int32, (TR, 1), 0)
    rows = jnp.where(rowid < N_USERS, rows, 0.0)
    tile = jnp.sum(rows, axis=0, keepdims=True) * (0.5 / N_USERS)

    @pl.when(i == 0)
    def _():
        out[...] = jnp.zeros((1, 1), jnp.float32)
    out[...] += tile


def _fusion_loss(e0_st, e1_st, ut_st, tmp_st, soc_st, f1_w1, f1_b1, f1_w2, f1_b2):
    n = NP // TR
    out = pl.pallas_call(
        _fusion_body,
        grid=(n,),
        in_specs=[
            pl.BlockSpec((2, TR, HALF), lambda i: (0, i, 0)),
            pl.BlockSpec((2, TR, HALF), lambda i: (0, i, 0)),
            pl.BlockSpec((2, TR, HALF), lambda i: (0, i, 0)),
            pl.BlockSpec((2, TR, HALF), lambda i: (0, i, 0)),
            pl.BlockSpec((2, TR, HALF), lambda i: (0, i, 0)),
            pl.BlockSpec((EMB, EMB), lambda i: (0, 0)),
            pl.BlockSpec((1, EMB), lambda i: (0, 0)),
            pl.BlockSpec((1, EMB), lambda i: (0, 0)),
            pl.BlockSpec((1, 1), lambda i: (0, 0)),
        ],
        out_specs=pl.BlockSpec((1, 1), lambda i: (0, 0)),
        out_shape=jax.ShapeDtypeStruct((1, 1), jnp.float32),
    )(e0_st, e1_st, ut_st, tmp_st, soc_st, f1_w1, f1_b1, f1_w2, f1_b2)
    return out[0, 0]


def kernel(social_hypergraph_list, cascade_hypergraph, examples, masks, lambda_u,
           rel_emb, rel_theta, rel_bias, cas_emb, cas_theta, cas_bias,
           infl_emb, co_attn_wts, f1_w1, f1_b1, f1_w2, f1_b2, soc_tab, tmp_tab):
    pad = ((0, NP - N_USERS), (0, 0))
    rel_p = jnp.pad(rel_emb, pad)
    cas_p = jnp.pad(cas_emb, pad)
    soc_p = jnp.pad(soc_tab, pad)
    tmp_p = jnp.pad(tmp_tab, pad)
    infl_p = jnp.pad(infl_emb, pad)

    xt_rel, xt_cas, g_st, ti_st, soc_st, tmp_st = _prep(
        rel_p, rel_theta, cas_p, cas_theta, soc_p, co_attn_wts, tmp_p, infl_p)

    idx_pad = jnp.full((EP - E_INC,), PAD_ROW, jnp.int32)

    def padi(x):
        return jnp.concatenate([x.astype(jnp.int32), idx_pad])

    ni0 = padi(social_hypergraph_list[0, 0])
    ei0 = padi(social_hypergraph_list[0, 1])
    ni1 = padi(social_hypergraph_list[1, 0])
    ei1 = padi(social_hypergraph_list[1, 1])
    nic = padi(cascade_hypergraph[0])
    eic = padi(cascade_hypergraph[1])

    xt_rel2 = xt_rel.reshape(2 * NP, HALF)
    xt_cas2 = xt_cas.reshape(2 * NP, HALF)

    onesr_c = jnp.ones((CH, HALF), jnp.float32)
    zrows_c = jnp.zeros((CH, HALF), jnp.float32)

    h0 = _deg_pass(ni0, ei0, onesr_c, zrows_c)
    h1 = _deg_pass(ni1, ei1, onesr_c, zrows_c)
    hc = _deg_pass(nic, eic, onesr_c, zrows_c)

    m0a = _segment_pass(xt_rel2, ni0, ei0, zrows_c)
    m1a = _segment_pass(xt_rel2, ni1, ei1, zrows_c)
    mca = _segment_pass(xt_cas2, nic, eic, zrows_c)

    d0, b0 = h0[:NP], h0[NP:]
    d1, b1d = h1[:NP], h1[NP:]
    dc, bc = hc[:NP], hc[NP:]

    m0 = _scale(m0a.reshape(2, NP, HALF), b0).reshape(2 * NP, HALF)
    m1 = _scale(m1a.reshape(2, NP, HALF), b1d).reshape(2 * NP, HALF)
    mc = _scale(mca.reshape(2, NP, HALF), bc).reshape(2 * NP, HALF)

    o0a = _segment_pass(m0, ei0, ni0, zrows_c)
    o1a = _segment_pass(m1, ei1, ni1, zrows_c)
    oca = _segment_pass(mc, eic, nic, zrows_c)

    rel_bias_st = rel_bias.reshape(2, 1, HALF)
    cas_bias_st = cas_bias.reshape(2, 1, HALF)
    e0_st = _scale_bias(o0a.reshape(2, NP, HALF), d0, rel_bias_st)
    e1_st = _scale_bias(o1a.reshape(2, NP, HALF), d1, rel_bias_st)
    ut_st = _scale_bias(oca.reshape(2, NP, HALF), dc, cas_bias_st)

    exf = examples.reshape(NTOK).astype(jnp.int32)
    grows, tirows = _example_gather(g_st.reshape(2 * NP, HALF),
                                    ti_st.reshape(2 * NP, HALF), exf)
    grows4 = grows.reshape(2, BATCH, SEQ, HALF)
    tirows4 = tirows.reshape(2, BATCH, SEQ, HALF)

    mf3 = masks.astype(jnp.float32).reshape(BATCH, SEQ, 1)
    pe = _make_pe(SEQ, EMB)
    pe_st = jnp.asarray(pe.reshape(SEQ, 2, HALF).transpose(1, 0, 2))

    h = _coattn(grows4, tirows4, mf3, pe_st)
    pred = _pred_matmul(h, soc_p)

    user_loss = lambda_u * _fusion_loss(
        e0_st, e1_st, ut_st, tmp_st, soc_st,
        f1_w1, f1_b1.reshape(1, EMB), f1_w2, f1_b2.reshape(1, 1))

    return (pred, co_attn_wts, user_loss)


# speedup vs baseline: 1.4497x; 1.4497x over previous
import functools

import jax
import jax.numpy as jnp
import numpy as np
from jax import lax
from jax.experimental import pallas as pl
from jax.experimental.pallas import tpu as pltpu
from jax.experimental.pallas import tpu_sc as plsc

N_USERS = 10000
NP = 10240
EMB = 256
HALF = 128
E_INC = 160000
CH = 128
NCHUNK = E_INC // CH
NTILE = 16
ZROWS = NP // NTILE
BATCH = 1024
SEQ = 50
NTOK = BATCH * SEQ
GCHUNKS = NTOK // CH


def _make_pe(L, d):
    position = np.arange(L)[:, None].astype(np.float32)
    div_term = np.exp(np.arange(0, d, 2).astype(np.float32) * -(np.log(10000.0) / d))
    pe = np.zeros((L, d), dtype=np.float32)
    pe[:, 0::2] = np.sin(position * div_term)
    pe[:, 1::2] = np.cos(position * div_term)
    return pe


def _seg_body(tab, gidx, sidx, zrows, out,
              gidx_v, sidx_v, rows_v, accum_sh, sem):
    cid = lax.axis_index("c")
    sid = lax.axis_index("s")

    pltpu.sync_copy(zrows, rows_v)
    for z in range(ZROWS // CH):
        start = sid * ZROWS + z * CH
        pltpu.sync_copy(rows_v, accum_sh.at[pl.ds(start, CH)])
    plsc.subcore_barrier()

    base_off = cid * NP
    count = jnp.where(sid < NCHUNK % NTILE, NCHUNK // NTILE + 1, NCHUNK // NTILE)

    def chunk(k, _):
        off = (sid + k * NTILE) * CH
        pltpu.sync_copy(gidx.at[pl.ds(off, CH)], gidx_v)
        pltpu.sync_copy(sidx.at[pl.ds(off, CH)], sidx_v)

        def prep(t, _):
            gv = gidx_v[pl.ds(t * 16, 16)]
            gidx_v[pl.ds(t * 16, 16)] = gv + base_off
            return 0
        lax.fori_loop(0, CH // 16, prep, 0)

        pltpu.async_copy(tab.at[gidx_v], rows_v, sem).wait()
        pltpu.sync_copy(rows_v, accum_sh.at[sidx_v], add=True)
        return 0
    lax.fori_loop(0, count, chunk, 0)

    plsc.subcore_barrier()
    for z in range(ZROWS // CH):
        start = sid * ZROWS + z * CH
        pltpu.sync_copy(accum_sh.at[pl.ds(start, CH)], rows_v)
        pltpu.sync_copy(rows_v, out.at[pl.ds(base_off + start, CH)])


def _segment_pass(tab2, gidx, sidx, zrows):
    mesh = plsc.VectorSubcoreMesh(core_axis_name="c", subcore_axis_name="s")
    f = pl.kernel(
        _seg_body,
        mesh=mesh,
        out_type=jax.ShapeDtypeStruct((2 * NP, HALF), jnp.float32),
        scratch_types=[
            pltpu.VMEM((CH,), jnp.int32),
            pltpu.VMEM((CH,), jnp.int32),
            pltpu.VMEM((CH, HALF), jnp.float32),
            pltpu.VMEM_SHARED((NP, HALF), jnp.float32),
            pltpu.SemaphoreType.DMA,
        ],
    )
    return f(tab2, gidx, sidx, zrows)


def _deg_body(gidx, sidx, onesr, zrows, out,
              idx_v, ones_v, rows_v, accum_sh, sem):
    cid = lax.axis_index("c")
    sid = lax.axis_index("s")

    pltpu.sync_copy(onesr, ones_v)
    pltpu.sync_copy(zrows, rows_v)
    for z in range(ZROWS // CH):
        start = sid * ZROWS + z * CH
        pltpu.sync_copy(rows_v, accum_sh.at[pl.ds(start, CH)])
    plsc.subcore_barrier()

    base_off = cid * NP
    count = jnp.where(sid < NCHUNK % NTILE, NCHUNK // NTILE + 1, NCHUNK // NTILE)

    def chunk(k, _):
        off = (sid + k * NTILE) * CH

        @pl.when(cid == 0)
        def _():
            pltpu.sync_copy(gidx.at[pl.ds(off, CH)], idx_v)

        @pl.when(cid == 1)
        def _():
            pltpu.sync_copy(sidx.at[pl.ds(off, CH)], idx_v)

        pltpu.sync_copy(ones_v, accum_sh.at[idx_v], add=True)
        return 0
    lax.fori_loop(0, count, chunk, 0)

    plsc.subcore_barrier()
    for z in range(ZROWS // CH):
        start = sid * ZROWS + z * CH
        pltpu.sync_copy(accum_sh.at[pl.ds(start, CH)], rows_v)
        pltpu.sync_copy(rows_v, out.at[pl.ds(base_off + start, CH)])


def _deg_pass(gidx, sidx, onesr, zrows):
    mesh = plsc.VectorSubcoreMesh(core_axis_name="c", subcore_axis_name="s")
    f = pl.kernel(
        _deg_body,
        mesh=mesh,
        out_type=jax.ShapeDtypeStruct((2 * NP, HALF), jnp.float32),
        scratch_types=[
            pltpu.VMEM((CH,), jnp.int32),
            pltpu.VMEM((CH, HALF), jnp.float32),
            pltpu.VMEM((CH, HALF), jnp.float32),
            pltpu.VMEM_SHARED((NP, HALF), jnp.float32),
            pltpu.SemaphoreType.DMA,
        ],
    )
    return f(gidx, sidx, onesr, zrows)


def _gath_body(gtab, titab, idx, grows, tirows, idx_v, rows_v, rows2_v, sem):
    cid = lax.axis_index("c")
    sid = lax.axis_index("s")
    base_off = cid * NP
    out_base = cid * NTOK

    def chunk(k, _):
        off = (sid + k * NTILE) * CH
        pltpu.sync_copy(idx.at[pl.ds(off, CH)], idx_v)

        def prep(t, _):
            idx_v[pl.ds(t * 16, 16)] = idx_v[pl.ds(t * 16, 16)] + base_off
            return 0
        lax.fori_loop(0, CH // 16, prep, 0)

        pltpu.async_copy(gtab.at[idx_v], rows_v, sem).wait()
        pltpu.sync_copy(rows_v, grows.at[pl.ds(out_base + off, CH)])
        pltpu.async_copy(titab.at[idx_v], rows2_v, sem).wait()
        pltpu.sync_copy(rows2_v, tirows.at[pl.ds(out_base + off, CH)])
        return 0
    lax.fori_loop(0, GCHUNKS // NTILE, chunk, 0)


def _example_gather(gtab2, titab2, idx):
    mesh = plsc.VectorSubcoreMesh(core_axis_name="c", subcore_axis_name="s")
    f = pl.kernel(
        _gath_body,
        mesh=mesh,
        out_type=[
            jax.ShapeDtypeStruct((2 * NTOK, HALF), jnp.float32),
            jax.ShapeDtypeStruct((2 * NTOK, HALF), jnp.float32),
        ],
        scratch_types=[
            pltpu.VMEM((CH,), jnp.int32),
            pltpu.VMEM((CH, HALF), jnp.float32),
            pltpu.VMEM((CH, HALF), jnp.float32),
            pltpu.SemaphoreType.DMA,
        ],
    )
    return f(gtab2, titab2, idx)


TR = 1024


def _prep_body(rel, rth, cas, cth, soc, caw, tmp, infl, soch, tmph,
               oxr, oxc, og, oti, osoc, otmp):
    oxr[0] = lax.dot_general(rel[...], rth[...], (((1,), (0,)), ((), ())),
                             preferred_element_type=jnp.float32)
    oxc[0] = lax.dot_general(cas[...], cth[...], (((1,), (0,)), ((), ())),
                             preferred_element_type=jnp.float32)
    og[0] = lax.dot_general(soc[...], caw[...], (((1,), (0,)), ((), ())),
                            preferred_element_type=jnp.float32)
    oti[0] = tmp[...] + infl[...]
    osoc[0] = soch[...]
    otmp[0] = tmph[...]


def _prep(rel_p, rel_theta, cas_p, cas_theta, soc_p, co_attn_wts, tmp_p, infl_p):
    n = NP // TR
    outs = pl.pallas_call(
        _prep_body,
        grid=(2, n),
        in_specs=[
            pl.BlockSpec((TR, EMB), lambda c, i: (i, 0)),
            pl.BlockSpec((EMB, HALF), lambda c, i: (0, c)),
            pl.BlockSpec((TR, EMB), lambda c, i: (i, 0)),
            pl.BlockSpec((EMB, HALF), lambda c, i: (0, c)),
            pl.BlockSpec((TR, EMB), lambda c, i: (i, 0)),
            pl.BlockSpec((EMB, HALF), lambda c, i: (0, c)),
            pl.BlockSpec((TR, HALF), lambda c, i: (i, c)),
            pl.BlockSpec((TR, HALF), lambda c, i: (i, c)),
            pl.BlockSpec((TR, HALF), lambda c, i: (i, c)),
            pl.BlockSpec((TR, HALF), lambda c, i: (i, c)),
        ],
        out_specs=[pl.BlockSpec((1, TR, HALF), lambda c, i: (c, i, 0))] * 6,
        out_shape=[jax.ShapeDtypeStruct((2, NP, HALF), jnp.float32)] * 6,
    )(rel_p, rel_theta, cas_p, cas_theta, soc_p, co_attn_wts, tmp_p, infl_p,
      soc_p, tmp_p)
    return outs


def _scale_body(acc, deg, out):
    d = deg[...]
    inv = jnp.where(d > 0, 1.0 / d, 0.0)
    out[0] = acc[0] * inv


def _scale(acc_st, deg128):
    n = NP // TR
    return pl.pallas_call(
        _scale_body,
        grid=(2, n),
        in_specs=[
            pl.BlockSpec((1, TR, HALF), lambda c, i: (c, i, 0)),
            pl.BlockSpec((TR, HALF), lambda c, i: (i, 0)),
        ],
        out_specs=pl.BlockSpec((1, TR, HALF), lambda c, i: (c, i, 0)),
        out_shape=jax.ShapeDtypeStruct((2, NP, HALF), jnp.float32),
    )(acc_st, deg128)


def _scale_bias_body(acc, deg, bias, out):
    d = deg[...]
    inv = jnp.where(d > 0, 1.0 / d, 0.0)
    out[0] = acc[0] * inv + bias[0]


def _scale_bias(acc_st, deg128, bias_st):
    n = NP // TR
    return pl.pallas_call(
        _scale_bias_body,
        grid=(2, n),
        in_specs=[
            pl.BlockSpec((1, TR, HALF), lambda c, i: (c, i, 0)),
            pl.BlockSpec((TR, HALF), lambda c, i: (i, 0)),
            pl.BlockSpec((1, 1, HALF), lambda c, i: (c, 0, 0)),
        ],
        out_specs=pl.BlockSpec((1, TR, HALF), lambda c, i: (c, i, 0)),
        out_shape=jax.ShapeDtypeStruct((2, NP, HALF), jnp.float32),
    )(acc_st, deg128, bias_st)


TB = 64


def _attn_body(g, ti, mf, pe, hout):
    gA, gB = g[0], g[1]
    tiA, tiB = ti[0], ti[1]
    mf3 = mf[...]
    mf2 = mf[:, :, 0]
    peA = jnp.broadcast_to(pe[0][None], (TB, SEQ, HALF))
    peB = jnp.broadcast_to(pe[1][None], (TB, SEQ, HALF))
    stA = tiA * mf3 + peA
    stB = tiB * mf3 + peB
    score = jnp.sum(gA * stA + gB * stB, axis=2) * mf2
    attn = jnp.tanh(score)
    m = jnp.max(attn, axis=1, keepdims=True)
    e = jnp.exp(attn - m)
    alpha = e / jnp.sum(e, axis=1, keepdims=True)
    a3 = alpha[:, :, None]
    hA = jnp.sum(stA * a3, axis=1)
    hB = jnp.sum(stB * a3, axis=1)
    hout[...] = jnp.concatenate([hA, hB], axis=1)


def _coattn(grows4, tirows4, mf3, pe_st):
    n = BATCH // TB
    return pl.pallas_call(
        _attn_body,
        grid=(n,),
        in_specs=[
            pl.BlockSpec((2, TB, SEQ, HALF), lambda i: (0, i, 0, 0)),
            pl.BlockSpec((2, TB, SEQ, HALF), lambda i: (0, i, 0, 0)),
            pl.BlockSpec((TB, SEQ, 1), lambda i: (i, 0, 0)),
            pl.BlockSpec((2, SEQ, HALF), lambda i: (0, 0, 0)),
        ],
        out_specs=pl.BlockSpec((TB, EMB), lambda i: (i, 0)),
        out_shape=jax.ShapeDtypeStruct((BATCH, EMB), jnp.float32),
    )(grows4, tirows4, mf3, pe_st)


def _pred_body(h_ref, soc_ref, out_ref):
    out_ref[...] = lax.dot_general(
        h_ref[...], soc_ref[...], (((1,), (1,)), ((), ())),
        preferred_element_type=jnp.float32)


def _pred_matmul(h, soc_p):
    TN = 1024
    out = pl.pallas_call(
        _pred_body,
        grid=(NP // TN,),
        in_specs=[
            pl.BlockSpec((BATCH, EMB), lambda i: (0, 0)),
            pl.BlockSpec((TN, EMB), lambda i: (i, 0)),
        ],
        out_specs=pl.BlockSpec((BATCH, TN), lambda i: (0, i)),
        out_shape=jax.ShapeDtypeStruct((BATCH, NP), jnp.float32),
    )(h, soc_p)
    return out[:, :N_USERS]


def _fusion_body(e0, e1, ut, tmp, soc, w1, b1, w2, b2, out):
    i = pl.program_id(0)
    e0A, e0B = e0[0], e0[1]
    e1A, e1B = e1[0], e1[1]
    w1A = w1[:, 0:HALF]
    w1B = w1[:, HALF:EMB]

    def head(eA, eB):
        z = (lax.dot_general(eA, w1A, (((1,), (1,)), ((), ())),
                             preferred_element_type=jnp.float32)
             + lax.dot_general(eB, w1B, (((1,), (1,)), ((), ())),
                               preferred_element_type=jnp.float32))
        t = jnp.tanh(z + b1[...])
        return jnp.sum(t * w2[...], axis=1, keepdims=True) + b2[...]

    w0 = head(e0A, e0B)
    w1s = head(e1A, e1B)
    gate = 2.0 * jax.nn.sigmoid(w0 - w1s)
    dA = ut[0] - tmp[0]
    dB = ut[1] - tmp[1]
    uA = gate * e0A - soc[0]
    uB = gate * e0B - soc[1]
    rows = jnp.sum(dA * dA + dB * dB + uA * uA + uB * uB, axis=1, keepdims=True)
    rowid = i * TR + lax.broadcasted_iota(jnp.int32, (TR, 1), 0)
    rows = jnp.where(rowid < N_USERS, rows, 0.0)
    tile = jnp.sum(rows, axis=0, keepdims=True) * (0.5 / N_USERS)

    @pl.when(i == 0)
    def _():
        out[...] = jnp.zeros((1, 1), jnp.float32)
    out[...] += tile


def _fusion_loss(e0_st, e1_st, ut_st, tmp_st, soc_st, f1_w1, f1_b1, f1_w2, f1_b2):
    n = NP // TR
    out = pl.pallas_call(
        _fusion_body,
        grid=(n,),
        in_specs=[
            pl.BlockSpec((2, TR, HALF), lambda i: (0, i, 0)),
            pl.BlockSpec((2, TR, HALF), lambda i: (0, i, 0)),
            pl.BlockSpec((2, TR, HALF), lambda i: (0, i, 0)),
            pl.BlockSpec((2, TR, HALF), lambda i: (0, i, 0)),
            pl.BlockSpec((2, TR, HALF), lambda i: (0, i, 0)),
            pl.BlockSpec((EMB, EMB), lambda i: (0, 0)),
            pl.BlockSpec((1, EMB), lambda i: (0, 0)),
            pl.BlockSpec((1, EMB), lambda i: (0, 0)),
            pl.BlockSpec((1, 1), lambda i: (0, 0)),
        ],
        out_specs=pl.BlockSpec((1, 1), lambda i: (0, 0)),
        out_shape=jax.ShapeDtypeStruct((1, 1), jnp.float32),
    )(e0_st, e1_st, ut_st, tmp_st, soc_st, f1_w1, f1_b1, f1_w2, f1_b2)
    return out[0, 0]


def kernel(social_hypergraph_list, cascade_hypergraph, examples, masks, lambda_u,
           rel_emb, rel_theta, rel_bias, cas_emb, cas_theta, cas_bias,
           infl_emb, co_attn_wts, f1_w1, f1_b1, f1_w2, f1_b2, soc_tab, tmp_tab):
    pad = ((0, NP - N_USERS), (0, 0))
    rel_p = jnp.pad(rel_emb, pad)
    cas_p = jnp.pad(cas_emb, pad)
    soc_p = jnp.pad(soc_tab, pad)
    tmp_p = jnp.pad(tmp_tab, pad)
    infl_p = jnp.pad(infl_emb, pad)

    xt_rel, xt_cas, g_st, ti_st, soc_st, tmp_st = _prep(
        rel_p, rel_theta, cas_p, cas_theta, soc_p, co_attn_wts, tmp_p, infl_p)

    ni0 = social_hypergraph_list[0, 0]
    ei0 = social_hypergraph_list[0, 1]
    ni1 = social_hypergraph_list[1, 0]
    ei1 = social_hypergraph_list[1, 1]
    nic = cascade_hypergraph[0]
    eic = cascade_hypergraph[1]

    xt_rel2 = xt_rel.reshape(2 * NP, HALF)
    xt_cas2 = xt_cas.reshape(2 * NP, HALF)

    onesr_c = jnp.ones((CH, HALF), jnp.float32)
    zrows_c = jnp.zeros((CH, HALF), jnp.float32)

    h0 = _deg_pass(ni0, ei0, onesr_c, zrows_c)
    h1 = _deg_pass(ni1, ei1, onesr_c, zrows_c)
    hc = _deg_pass(nic, eic, onesr_c, zrows_c)

    m0a = _segment_pass(xt_rel2, ni0, ei0, zrows_c)
    m1a = _segment_pass(xt_rel2, ni1, ei1, zrows_c)
    mca = _segment_pass(xt_cas2, nic, eic, zrows_c)

    d0, b0 = h0[:NP], h0[NP:]
    d1, b1d = h1[:NP], h1[NP:]
    dc, bc = hc[:NP], hc[NP:]

    m0 = _scale(m0a.reshape(2, NP, HALF), b0).reshape(2 * NP, HALF)
    m1 = _scale(m1a.reshape(2, NP, HALF), b1d).reshape(2 * NP, HALF)
    mc = _scale(mca.reshape(2, NP, HALF), bc).reshape(2 * NP, HALF)

    o0a = _segment_pass(m0, ei0, ni0, zrows_c)
    o1a = _segment_pass(m1, ei1, ni1, zrows_c)
    oca = _segment_pass(mc, eic, nic, zrows_c)

    rel_bias_st = rel_bias.reshape(2, 1, HALF)
    cas_bias_st = cas_bias.reshape(2, 1, HALF)
    e0_st = _scale_bias(o0a.reshape(2, NP, HALF), d0, rel_bias_st)
    e1_st = _scale_bias(o1a.reshape(2, NP, HALF), d1, rel_bias_st)
    ut_st = _scale_bias(oca.reshape(2, NP, HALF), dc, cas_bias_st)

    exf = examples.reshape(NTOK).astype(jnp.int32)
    grows, tirows = _example_gather(g_st.reshape(2 * NP, HALF),
                                    ti_st.reshape(2 * NP, HALF), exf)
    grows4 = grows.reshape(2, BATCH, SEQ, HALF)
    tirows4 = tirows.reshape(2, BATCH, SEQ, HALF)

    mf3 = masks.astype(jnp.float32).reshape(BATCH, SEQ, 1)
    pe = _make_pe(SEQ, EMB)
    pe_st = jnp.asarray(pe.reshape(SEQ, 2, HALF).transpose(1, 0, 2))

    h = _coattn(grows4, tirows4, mf3, pe_st)
    pred = _pred_matmul(h, soc_p)

    user_loss = lambda_u * _fusion_loss(
        e0_st, e1_st, ut_st, tmp_st, soc_st,
        f1_w1, f1_b1.reshape(1, EMB), f1_w2, f1_b2.reshape(1, 1))

    return (pred, co_attn_wts, user_loss)
